# Initial kernel scaffold; baseline (speedup 1.0000x reference)
import jax
import jax.numpy as jnp
from jax.experimental import pallas as pl


def kernel(features, positions, mask, W, b):
    raise NotImplementedError("write your pallas kernel here")


# speedup vs baseline: 1.3923x; 1.3923x over previous
import functools

import jax
import jax.numpy as jnp
import numpy as np
from jax import lax
from jax.experimental import pallas as pl
from jax.experimental.pallas import tpu as pltpu
from jax.experimental.pallas import tpu_sc as plsc

B = 32
N = 8192
D = 256
P3 = 3
K = 500
KPAD = 512
NVREG = N // 16
CHUNK = 64

_MAG = np.int32(0x7FFFFFFF)
_NEG_INF_KEY = np.int32(-2139095041)
_PAD_KEY = np.int32(-2147483648)

BUDGET = 1024

STRIDE = 16
DOFF = 16
OFS_CI = 20480
SLAB = 40960


def _lane():
    return lax.broadcasted_iota(jnp.int32, (16,), 0)


def _pget(v, idx):
    return v.at[idx].get(mode="promise_in_bounds", unique_indices=True)


def _pget_dup(v, idx):
    return v.at[idx].get(mode="promise_in_bounds")


def _hillis(v, ln):
    for d in (1, 2, 4, 8):
        sh = _pget_dup(v, jnp.maximum(ln - d, 0))
        v = v + jnp.where(ln >= d, sh, 0)
    return v


def _cmp_gt(ka, ia, kb, ib):
    return (ka > kb) | ((ka == kb) & (ia < ib))


def _i2f(x):
    return lax.bitcast_convert_type(x, jnp.float32)


def _f2i(x):
    return lax.bitcast_convert_type(x, jnp.int32)


def _sc_body(scores_hbm, feat_hbm, pos_hbm,
             feat_out, pos_out, scr_out,
             sc_v, key_v, kf_v, if_v, dest_v,
             cb_f, cand_k, cand_i,
             idxg_v, idxp_v, fchunk_v, fchunk2_v, pout_v, scrv_v,
             cnt_s, shp, sem):
    cid = lax.axis_index("c")
    sid = lax.axis_index("s")
    b = sid * 2 + cid
    sbase = sid * SLAB
    ln = _lane()

    pltpu.sync_copy(scores_hbm.at[b], sc_v)

    def cvt(i, cnt):
        f = sc_v[pl.ds(i * 16, 16)]
        bi = _f2i(f)
        k = bi ^ ((bi >> 31) & _MAG)
        key_v[pl.ds(i * 16, 16)] = k
        kf_v[pl.ds(i * 16, 16)] = _i2f(k)
        if_v[pl.ds(i * 16, 16)] = _i2f(ln + i * 16)
        return cnt + jnp.where(k != _NEG_INF_KEY, 1, 0)

    vcnt = lax.fori_loop(0, NVREG, cvt, jnp.zeros((16,), jnp.int32))
    for d in (8, 4, 2, 1):
        vcnt = vcnt + _pget(vcnt, ln ^ d)
    valid = vcnt[0]
    kval = jnp.minimum(jnp.maximum(valid, 10), K)

    def count_gt(T):
        def cg(i, acc):
            k = key_v[pl.ds(i * 16, 16)]
            return acc + jnp.where(k > T, 1, 0)
        acc = lax.fori_loop(0, NVREG, cg, jnp.zeros((16,), jnp.int32))
        for d in (8, 4, 2, 1):
            acc = acc + _pget(acc, ln ^ d)
        return acc[0]

    def bs_body(it, st):
        lo, hi, T, found = st
        act = (found == 0) & (hi > lo + 1)
        m = (lo >> 1) + (hi >> 1) + (lo & hi & 1)

        @pl.when(act)
        def _count():
            cnt_s[0] = count_gt(m)

        c = cnt_s[0]
        found2 = act & (c >= K) & (c <= BUDGET)
        T = jnp.where(found2, m, T)
        lo = jnp.where(act & (c > BUDGET), m, lo)
        hi = jnp.where(act & (c < K), m, hi)
        return (lo, hi, T, found | jnp.where(found2, 1, 0))

    lo, hi, T, found = lax.fori_loop(
        0, 32, bs_body,
        (jnp.int32(-2147483648), jnp.int32(2147483647),
         jnp.int32(0), jnp.int32(0)))
    kstar = jnp.where(found == 1, T, hi)
    c1 = count_gt(kstar)
    need = jnp.where(found == 1, 0, K - c1)
    c_sel = c1 + need

    def comp(i, off):
        k = key_v[pl.ds(i * 16, 16)]
        e = ln + i * 16
        m_gt = k > kstar
        m_eq = k == kstar
        packed = jnp.where(m_gt, 65536, 0) + jnp.where(m_eq, 1, 0)
        incl = _hillis(packed, ln) + off
        r_gt = (incl >> 16) - 1
        r_eq = (incl & 0xFFFF) - 1
        dead = BUDGET + (e & 0xFF)
        dl = jnp.where(
            m_gt, r_gt,
            jnp.where(m_eq & (r_eq < need), c1 + r_eq, dead))
        dest_v[pl.ds(i * 16, 16)] = sbase + dl * STRIDE
        return incl[15]

    lax.fori_loop(0, NVREG, comp, jnp.int32(0))
    pltpu.sync_copy(kf_v, shp.at[dest_v])

    def shift_ci(i, _):
        dest_v[pl.ds(i * 16, 16)] = dest_v[pl.ds(i * 16, 16)] + OFS_CI
        return 0
    lax.fori_loop(0, NVREG, shift_ci, 0)
    pltpu.sync_copy(if_v, shp.at[dest_v])

    for half, ofs in ((0, 0), (1, OFS_CI)):
        for cc in range(BUDGET * STRIDE // 2048):
            pltpu.sync_copy(
                shp.at[pl.ds(sbase + ofs + cc * 2048, 2048)],
                cb_f.at[pl.ds(0, 2048)])

            def diag(j, _, cc=cc, half=half):
                acc = jnp.zeros((16,), jnp.int32)
                for t in range(16):
                    v = _f2i(cb_f[pl.ds(j * 256 + t * 16, 16)])
                    acc = acc + jnp.where(ln == t,
                                          jnp.broadcast_to(v[0], (16,)), 0)
                pos = ln + cc * 128 + j * 16
                live = pos < c_sel
                if half == 0:
                    cand_k[pl.ds(DOFF + cc * 128 + j * 16, 16)] = jnp.where(
                        live, acc, _PAD_KEY)
                else:
                    cand_i[pl.ds(DOFF + cc * 128 + j * 16, 16)] = jnp.where(
                        live, acc, N + (pos & 31))
                return 0
            lax.fori_loop(0, 8, diag, 0)

    for klog in range(1, 11):
        for dlog in range(klog - 1, -1, -1):
            dd = 1 << dlog
            if dlog >= 4:
                h = dd >> 4

                def cross(t, _, h=h, klog=klog):
                    m = ((t & ~(h - 1)) << 1) | (t & (h - 1))
                    p = m + h
                    ka = cand_k[pl.ds(DOFF + m * 16, 16)]
                    ia = cand_i[pl.ds(DOFF + m * 16, 16)]
                    kb = cand_k[pl.ds(DOFF + p * 16, 16)]
                    ib = cand_i[pl.ds(DOFF + p * 16, 16)]
                    di = ((m * 16 + ln) >> klog) & 1
                    c = _cmp_gt(ka, ia, kb, ib)
                    ta = jnp.where(c, 1, 0) == (1 - di)
                    cand_k[pl.ds(DOFF + m * 16, 16)] = jnp.where(ta, ka, kb)
                    cand_i[pl.ds(DOFF + m * 16, 16)] = jnp.where(ta, ia, ib)
                    cand_k[pl.ds(DOFF + p * 16, 16)] = jnp.where(ta, kb, ka)
                    cand_i[pl.ds(DOFF + p * 16, 16)] = jnp.where(ta, ib, ia)
                    return 0
                lax.fori_loop(0, BUDGET // 32, cross, 0)
            else:
                def intra(m, _, dd=dd, klog=klog):
                    base = DOFF + m * 16
                    k = cand_k[pl.ds(base, 16)]
                    i_ = cand_i[pl.ds(base, 16)]
                    up_sel = (ln & dd) == 0
                    pk = jnp.where(up_sel,
                                   cand_k[pl.ds(base + dd, 16)],
                                   cand_k[pl.ds(base - dd, 16)])
                    pi = jnp.where(up_sel,
                                   cand_i[pl.ds(base + dd, 16)],
                                   cand_i[pl.ds(base - dd, 16)])
                    di = ((m * 16 + ln) >> klog) & 1
                    lowi = jnp.where(up_sel, 1, 0)
                    c = _cmp_gt(k, i_, pk, pi)
                    wi = jnp.where(lowi == (1 - di), 1, 0)
                    ts = jnp.where(c, 1, 0) == wi
                    cand_k[pl.ds(base, 16)] = jnp.where(ts, k, pk)
                    cand_i[pl.ds(base, 16)] = jnp.where(ts, i_, pi)
                    return 0
                lax.fori_loop(0, BUDGET // 16, intra, 0)

    def so(j, _):
        k = cand_k[pl.ds(DOFF + j * 16, 16)]
        bits = k ^ ((k >> 31) & _MAG)
        f = lax.bitcast_convert_type(bits, jnp.float32)
        sig = 1.0 / (1.0 + jnp.exp(-f))
        slot = ln + j * 16
        scrv_v[pl.ds(j * 16, 16)] = jnp.where(slot < kval, sig, 0.0)
        return 0
    lax.fori_loop(0, KPAD // 16, so, 0)
    pltpu.sync_copy(scrv_v, scr_out.at[b])

    base = b * N
    pbase = b * (P3 * N)

    def gidx(j, _):
        ii = jnp.minimum(cand_i[pl.ds(DOFF + j * 16, 16)], N - 1)
        idxg_v[pl.ds(j * 16, 16)] = ii + base
        for p in range(P3):
            idxp_v[pl.ds(p * KPAD + j * 16, 16)] = ii + (pbase + p * N)
        return 0
    lax.fori_loop(0, KPAD // 16, gidx, 0)

    for p in range(P3):
        for q in range(KPAD // 128):
            o = p * KPAD + q * 128
            pltpu.async_copy(pos_hbm.at[idxp_v.at[pl.ds(o, 128)]],
                             pout_v.at[pl.ds(o, 128)], sem).wait()

    def pmask(j, _):
        slot = ln + (j % (KPAD // 16)) * 16
        v = pout_v[pl.ds(j * 16, 16)]
        pout_v[pl.ds(j * 16, 16)] = jnp.where(slot < kval, v, 0.0)
        return 0
    lax.fori_loop(0, P3 * KPAD // 16, pmask, 0)
    pltpu.sync_copy(pout_v, pos_out.at[b])

    for ci_ in range(KPAD // CHUNK):
        start = ci_ * CHUNK
        buf = fchunk_v if ci_ % 2 == 0 else fchunk2_v
        pltpu.async_copy(feat_hbm.at[idxg_v.at[pl.ds(start, CHUNK)]],
                         buf, sem).wait()

        @pl.when(kval < K)
        def _zero_tail(buf=buf, start=start):
            def zr(r, _):
                m = jnp.where(start + r < kval, 1.0, 0.0)
                def zcol(l, _):
                    fv = buf[r, pl.ds(l * 16, 16)]
                    buf[r, pl.ds(l * 16, 16)] = fv * m
                    return 0
                lax.fori_loop(0, D // 16, zcol, 0)
                return 0
            lax.fori_loop(0, CHUNK, zr, 0)

        pltpu.sync_copy(buf, feat_out.at[b, pl.ds(start, CHUNK)])


def _run_sc(scores, feat2d, posT):
    mesh = plsc.VectorSubcoreMesh(core_axis_name="c", subcore_axis_name="s")
    f = functools.partial(
        pl.kernel,
        mesh=mesh,
        out_type=[
            jax.ShapeDtypeStruct((B, KPAD, D), jnp.float32),
            jax.ShapeDtypeStruct((B, P3 * KPAD), jnp.float32),
            jax.ShapeDtypeStruct((B, KPAD), jnp.float32),
        ],
        scratch_types=[
            pltpu.VMEM((N,), jnp.float32),
            pltpu.VMEM((N,), jnp.int32),
            pltpu.VMEM((N,), jnp.float32),
            pltpu.VMEM((N,), jnp.float32),
            pltpu.VMEM((N,), jnp.int32),
            pltpu.VMEM((2048,), jnp.float32),
            pltpu.VMEM((BUDGET + 32,), jnp.int32),
            pltpu.VMEM((BUDGET + 32,), jnp.int32),
            pltpu.VMEM((KPAD,), jnp.int32),
            pltpu.VMEM((P3 * KPAD,), jnp.int32),
            pltpu.VMEM((CHUNK, D), jnp.float32),
            pltpu.VMEM((CHUNK, D), jnp.float32),
            pltpu.VMEM((P3 * KPAD,), jnp.float32),
            pltpu.VMEM((KPAD,), jnp.float32),
            pltpu.SMEM((16,), jnp.int32),
            pltpu.VMEM_SHARED((16 * SLAB,), jnp.float32),
            pltpu.SemaphoreType.DMA,
        ],
    )(_sc_body)
    return f(scores, feat2d, posT)


def kernel(features, positions, mask, W, b):
    scores = jnp.where(mask, (features @ W + b).squeeze(-1),
                       jnp.float32(-jnp.inf))
    feat2d = features.reshape(B * N, D)
    posT = jnp.transpose(positions, (0, 2, 1)).reshape(B * P3 * N)
    feat_res, pos_res, scr_res = _run_sc(scores, feat2d, posT)
    padded_pos = jnp.transpose(pos_res.reshape(B, P3, KPAD), (0, 2, 1))[:, :K]
    padded_scores = scr_res[:, :K]
    return (padded_pos, feat_res[:, :K], padded_scores)


# speedup vs baseline: 1.4673x; 1.0539x over previous
import functools

import jax
import jax.numpy as jnp
import numpy as np
from jax import lax
from jax.experimental import pallas as pl
from jax.experimental.pallas import tpu as pltpu
from jax.experimental.pallas import tpu_sc as plsc

B = 32
N = 8192
D = 256
P3 = 3
K = 500
KPAD = 512
NVREG = N // 16
CHUNK = 64

_MAG = np.int32(0x7FFFFFFF)
_NEG_INF_KEY = np.int32(-2139095041)
_PAD_KEY = np.int32(-2147483648)

BUDGET = 1024

STRIDE = 16
DOFF = 16
OFS_CI = 20480
SLAB = 40960


def _lane():
    return lax.broadcasted_iota(jnp.int32, (16,), 0)


def _pget(v, idx):
    return v.at[idx].get(mode="promise_in_bounds", unique_indices=True)


def _pget_dup(v, idx):
    return v.at[idx].get(mode="promise_in_bounds")


def _hillis(v, ln):
    for d in (1, 2, 4, 8):
        sh = _pget_dup(v, jnp.maximum(ln - d, 0))
        v = v + jnp.where(ln >= d, sh, 0)
    return v


def _cmp_gt(ka, ia, kb, ib):
    return (ka > kb) | ((ka == kb) & (ia < ib))


def _i2f(x):
    return lax.bitcast_convert_type(x, jnp.float32)


def _f2i(x):
    return lax.bitcast_convert_type(x, jnp.int32)


def _sc_body(scores_hbm, feat_hbm, pos_hbm,
             feat_out, pos_out, scr_out,
             sc_v, key_v, kf_v, if_v, dest_v,
             cb_f, cand_k, cand_i,
             idxg_v, idxp_v, fchunk_v, fchunk2_v, pout_v, scrv_v,
             cnt_s, shp, sem):
    cid = lax.axis_index("c")
    sid = lax.axis_index("s")
    b = sid * 2 + cid
    sbase = sid * SLAB
    ln = _lane()

    pltpu.sync_copy(scores_hbm.at[b], sc_v)

    def cvt(i, cnt):
        f = sc_v[pl.ds(i * 16, 16)]
        bi = _f2i(f)
        k = bi ^ ((bi >> 31) & _MAG)
        key_v[pl.ds(i * 16, 16)] = k
        kf_v[pl.ds(i * 16, 16)] = _i2f(k)
        if_v[pl.ds(i * 16, 16)] = _i2f(ln + i * 16)
        return cnt + jnp.where(k != _NEG_INF_KEY, 1, 0)

    vcnt = lax.fori_loop(0, NVREG, cvt, jnp.zeros((16,), jnp.int32))
    for d in (8, 4, 2, 1):
        vcnt = vcnt + _pget(vcnt, ln ^ d)
    valid = vcnt[0]
    kval = jnp.minimum(jnp.maximum(valid, 10), K)

    def count_gt(T):
        def cg(i, acc):
            a = acc
            for u in range(4):
                k = key_v[pl.ds(i * 64 + u * 16, 16)]
                a = a + jnp.where(k > T, 1, 0)
            return a
        acc = lax.fori_loop(0, NVREG // 4, cg, jnp.zeros((16,), jnp.int32))
        for d in (8, 4, 2, 1):
            acc = acc + _pget(acc, ln ^ d)
        return acc[0]

    def bs_body(it, st):
        lo, hi, T, found = st
        act = (found == 0) & (hi > lo + 1)
        m = (lo >> 1) + (hi >> 1) + (lo & hi & 1)

        @pl.when(act)
        def _count():
            cnt_s[0] = count_gt(m)

        c = cnt_s[0]
        found2 = act & (c >= K) & (c <= BUDGET)
        T = jnp.where(found2, m, T)
        lo = jnp.where(act & (c > BUDGET), m, lo)
        hi = jnp.where(act & (c < K), m, hi)
        return (lo, hi, T, found | jnp.where(found2, 1, 0))

    lo, hi, T, found = lax.fori_loop(
        0, 32, bs_body,
        (jnp.int32(-2147483648), jnp.int32(2147483647),
         jnp.int32(0), jnp.int32(0)))
    kstar = jnp.where(found == 1, T, hi)
    c1 = count_gt(kstar)
    need = jnp.where(found == 1, 0, K - c1)
    c_sel = c1 + need

    def comp(i, off):
        k = key_v[pl.ds(i * 16, 16)]
        e = ln + i * 16
        m_gt = k > kstar
        m_eq = k == kstar
        packed = jnp.where(m_gt, 65536, 0) + jnp.where(m_eq, 1, 0)
        incl = _hillis(packed, ln) + off
        r_gt = (incl >> 16) - 1
        r_eq = (incl & 0xFFFF) - 1
        dead = BUDGET + (e & 0xFF)
        dl = jnp.where(
            m_gt, r_gt,
            jnp.where(m_eq & (r_eq < need), c1 + r_eq, dead))
        dest_v[pl.ds(i * 16, 16)] = sbase + dl * STRIDE
        return incl[15]

    lax.fori_loop(0, NVREG, comp, jnp.int32(0))
    pltpu.sync_copy(kf_v, shp.at[dest_v])

    def shift_ci(i, _):
        dest_v[pl.ds(i * 16, 16)] = dest_v[pl.ds(i * 16, 16)] + OFS_CI
        return 0
    lax.fori_loop(0, NVREG, shift_ci, 0)
    pltpu.sync_copy(if_v, shp.at[dest_v])

    for half, ofs in ((0, 0), (1, OFS_CI)):
        for cc in range(BUDGET * STRIDE // 2048):
            pltpu.sync_copy(
                shp.at[pl.ds(sbase + ofs + cc * 2048, 2048)],
                cb_f.at[pl.ds(0, 2048)])

            def diag(j, _, cc=cc, half=half):
                acc = jnp.zeros((16,), jnp.int32)
                for t in range(16):
                    v = _f2i(cb_f[pl.ds(j * 256 + t * 16, 16)])
                    acc = acc + jnp.where(ln == t,
                                          jnp.broadcast_to(v[0], (16,)), 0)
                pos = ln + cc * 128 + j * 16
                live = pos < c_sel
                if half == 0:
                    cand_k[pl.ds(DOFF + cc * 128 + j * 16, 16)] = jnp.where(
                        live, acc, _PAD_KEY)
                else:
                    cand_i[pl.ds(DOFF + cc * 128 + j * 16, 16)] = jnp.where(
                        live, acc, N + (pos & 31))
                return 0
            lax.fori_loop(0, 8, diag, 0)

    for klog in range(1, 11):
        for dlog in range(klog - 1, -1, -1):
            dd = 1 << dlog
            if dlog >= 4:
                h = dd >> 4

                def cross(t, _, h=h, klog=klog):
                    m = ((t & ~(h - 1)) << 1) | (t & (h - 1))
                    p = m + h
                    ka = cand_k[pl.ds(DOFF + m * 16, 16)]
                    ia = cand_i[pl.ds(DOFF + m * 16, 16)]
                    kb = cand_k[pl.ds(DOFF + p * 16, 16)]
                    ib = cand_i[pl.ds(DOFF + p * 16, 16)]
                    di = ((m * 16 + ln) >> klog) & 1
                    c = _cmp_gt(ka, ia, kb, ib)
                    ta = jnp.where(c, 1, 0) == (1 - di)
                    cand_k[pl.ds(DOFF + m * 16, 16)] = jnp.where(ta, ka, kb)
                    cand_i[pl.ds(DOFF + m * 16, 16)] = jnp.where(ta, ia, ib)
                    cand_k[pl.ds(DOFF + p * 16, 16)] = jnp.where(ta, kb, ka)
                    cand_i[pl.ds(DOFF + p * 16, 16)] = jnp.where(ta, ib, ia)
                    return 0
                lax.fori_loop(0, BUDGET // 32, cross, 0)
            else:
                def intra(m, _, dd=dd, klog=klog):
                    base = DOFF + m * 16
                    k = cand_k[pl.ds(base, 16)]
                    i_ = cand_i[pl.ds(base, 16)]
                    up_sel = (ln & dd) == 0
                    pk = jnp.where(up_sel,
                                   cand_k[pl.ds(base + dd, 16)],
                                   cand_k[pl.ds(base - dd, 16)])
                    pi = jnp.where(up_sel,
                                   cand_i[pl.ds(base + dd, 16)],
                                   cand_i[pl.ds(base - dd, 16)])
                    di = ((m * 16 + ln) >> klog) & 1
                    lowi = jnp.where(up_sel, 1, 0)
                    c = _cmp_gt(k, i_, pk, pi)
                    wi = jnp.where(lowi == (1 - di), 1, 0)
                    ts = jnp.where(c, 1, 0) == wi
                    cand_k[pl.ds(base, 16)] = jnp.where(ts, k, pk)
                    cand_i[pl.ds(base, 16)] = jnp.where(ts, i_, pi)
                    return 0
                lax.fori_loop(0, BUDGET // 16, intra, 0)

    def so(j, _):
        k = cand_k[pl.ds(DOFF + j * 16, 16)]
        bits = k ^ ((k >> 31) & _MAG)
        f = lax.bitcast_convert_type(bits, jnp.float32)
        sig = 1.0 / (1.0 + jnp.exp(-f))
        slot = ln + j * 16
        scrv_v[pl.ds(j * 16, 16)] = jnp.where(slot < kval, sig, 0.0)
        return 0
    lax.fori_loop(0, KPAD // 16, so, 0)
    pltpu.sync_copy(scrv_v, scr_out.at[b])

    base = b * N
    pbase = b * (P3 * N)

    def gidx(j, _):
        ii = jnp.minimum(cand_i[pl.ds(DOFF + j * 16, 16)], N - 1)
        idxg_v[pl.ds(j * 16, 16)] = ii + base
        for p in range(P3):
            idxp_v[pl.ds(p * KPAD + j * 16, 16)] = ii + (pbase + p * N)
        return 0
    lax.fori_loop(0, KPAD // 16, gidx, 0)

    for p in range(P3):
        for q in range(KPAD // 128):
            o = p * KPAD + q * 128
            pltpu.async_copy(pos_hbm.at[idxp_v.at[pl.ds(o, 128)]],
                             pout_v.at[pl.ds(o, 128)], sem).wait()

    def pmask(j, _):
        slot = ln + (j % (KPAD // 16)) * 16
        v = pout_v[pl.ds(j * 16, 16)]
        pout_v[pl.ds(j * 16, 16)] = jnp.where(slot < kval, v, 0.0)
        return 0
    lax.fori_loop(0, P3 * KPAD // 16, pmask, 0)
    pltpu.sync_copy(pout_v, pos_out.at[b])

    for ci_ in range(KPAD // CHUNK):
        start = ci_ * CHUNK
        buf = fchunk_v if ci_ % 2 == 0 else fchunk2_v
        pltpu.async_copy(feat_hbm.at[idxg_v.at[pl.ds(start, CHUNK)]],
                         buf, sem).wait()

        @pl.when(kval < K)
        def _zero_tail(buf=buf, start=start):
            def zr(r, _):
                m = jnp.where(start + r < kval, 1.0, 0.0)
                def zcol(l, _):
                    fv = buf[r, pl.ds(l * 16, 16)]
                    buf[r, pl.ds(l * 16, 16)] = fv * m
                    return 0
                lax.fori_loop(0, D // 16, zcol, 0)
                return 0
            lax.fori_loop(0, CHUNK, zr, 0)

        pltpu.sync_copy(buf, feat_out.at[b, pl.ds(start, CHUNK)])


def _run_sc(scores, feat2d, posT):
    mesh = plsc.VectorSubcoreMesh(core_axis_name="c", subcore_axis_name="s")
    f = functools.partial(
        pl.kernel,
        mesh=mesh,
        out_type=[
            jax.ShapeDtypeStruct((B, KPAD, D), jnp.float32),
            jax.ShapeDtypeStruct((B, P3 * KPAD), jnp.float32),
            jax.ShapeDtypeStruct((B, KPAD), jnp.float32),
        ],
        scratch_types=[
            pltpu.VMEM((N,), jnp.float32),
            pltpu.VMEM((N,), jnp.int32),
            pltpu.VMEM((N,), jnp.float32),
            pltpu.VMEM((N,), jnp.float32),
            pltpu.VMEM((N,), jnp.int32),
            pltpu.VMEM((2048,), jnp.float32),
            pltpu.VMEM((BUDGET + 32,), jnp.int32),
            pltpu.VMEM((BUDGET + 32,), jnp.int32),
            pltpu.VMEM((KPAD,), jnp.int32),
            pltpu.VMEM((P3 * KPAD,), jnp.int32),
            pltpu.VMEM((CHUNK, D), jnp.float32),
            pltpu.VMEM((CHUNK, D), jnp.float32),
            pltpu.VMEM((P3 * KPAD,), jnp.float32),
            pltpu.VMEM((KPAD,), jnp.float32),
            pltpu.SMEM((16,), jnp.int32),
            pltpu.VMEM_SHARED((16 * SLAB,), jnp.float32),
            pltpu.SemaphoreType.DMA,
        ],
    )(_sc_body)
    return f(scores, feat2d, posT)


def kernel(features, positions, mask, W, b):
    scores = jnp.where(mask, (features @ W + b).squeeze(-1),
                       jnp.float32(-jnp.inf))
    feat2d = features.reshape(B * N, D)
    posT = jnp.transpose(positions, (0, 2, 1)).reshape(B * P3 * N)
    feat_res, pos_res, scr_res = _run_sc(scores, feat2d, posT)
    padded_pos = jnp.transpose(pos_res.reshape(B, P3, KPAD), (0, 2, 1))[:, :K]
    padded_scores = scr_res[:, :K]
    return (padded_pos, feat_res[:, :K], padded_scores)


# speedup vs baseline: 1.4794x; 1.0083x over previous
import functools

import jax
import jax.numpy as jnp
import numpy as np
from jax import lax
from jax.experimental import pallas as pl
from jax.experimental.pallas import tpu as pltpu
from jax.experimental.pallas import tpu_sc as plsc

B = 32
N = 8192
D = 256
P3 = 3
K = 500
KPAD = 512
NVREG = N // 16
CHUNK = 64

_MAG = np.int32(0x7FFFFFFF)
_NEG_INF_KEY = np.int32(-2139095041)
_PAD_KEY = np.int32(-2147483648)

BUDGET = 1024

STRIDE = 16
DOFF = 16
OFS_CI = 20480
SLAB = 40960


def _lane():
    return lax.broadcasted_iota(jnp.int32, (16,), 0)


def _pget(v, idx):
    return v.at[idx].get(mode="promise_in_bounds", unique_indices=True)


def _pget_dup(v, idx):
    return v.at[idx].get(mode="promise_in_bounds")


def _hillis(v, ln):
    for d in (1, 2, 4, 8):
        sh = _pget_dup(v, jnp.maximum(ln - d, 0))
        v = v + jnp.where(ln >= d, sh, 0)
    return v


def _cmp_gt(ka, ia, kb, ib):
    return (ka > kb) | ((ka == kb) & (ia < ib))


def _i2f(x):
    return lax.bitcast_convert_type(x, jnp.float32)


def _f2i(x):
    return lax.bitcast_convert_type(x, jnp.int32)


def _sc_body(scores_hbm, feat_hbm, pos_hbm,
             feat_out, pos_out, scr_out,
             sc_v, key_v, kf_v, if_v, dest_v,
             cb_f, cand_k, cand_i,
             idxg_v, idxp_v, fchunk_v, fchunk2_v, pout_v, scrv_v,
             cnt_s, shp, sem):
    cid = lax.axis_index("c")
    sid = lax.axis_index("s")
    b = sid * 2 + cid
    sbase = sid * SLAB
    ln = _lane()

    pltpu.sync_copy(scores_hbm.at[b], sc_v)

    def cvt(i, cnt):
        for u in range(2):
            o = i * 32 + u * 16
            f = sc_v[pl.ds(o, 16)]
            bi = _f2i(f)
            k = bi ^ ((bi >> 31) & _MAG)
            key_v[pl.ds(o, 16)] = k
            kf_v[pl.ds(o, 16)] = _i2f(k)
            if_v[pl.ds(o, 16)] = _i2f(ln + o)
            cnt = cnt + jnp.where(k != _NEG_INF_KEY, 1, 0)
        return cnt

    vcnt = lax.fori_loop(0, NVREG // 2, cvt, jnp.zeros((16,), jnp.int32))
    for d in (8, 4, 2, 1):
        vcnt = vcnt + _pget(vcnt, ln ^ d)
    valid = vcnt[0]
    kval = jnp.minimum(jnp.maximum(valid, 10), K)

    def count_gt(T):
        def cg(i, acc):
            a = acc
            for u in range(4):
                k = key_v[pl.ds(i * 64 + u * 16, 16)]
                a = a + jnp.where(k > T, 1, 0)
            return a
        acc = lax.fori_loop(0, NVREG // 4, cg, jnp.zeros((16,), jnp.int32))
        for d in (8, 4, 2, 1):
            acc = acc + _pget(acc, ln ^ d)
        return acc[0]

    def bs_body(it, st):
        lo, hi, T, found = st
        act = (found == 0) & (hi > lo + 1)
        m = (lo >> 1) + (hi >> 1) + (lo & hi & 1)

        @pl.when(act)
        def _count():
            cnt_s[0] = count_gt(m)

        c = cnt_s[0]
        found2 = act & (c >= K) & (c <= BUDGET)
        T = jnp.where(found2, m, T)
        lo = jnp.where(act & (c > BUDGET), m, lo)
        hi = jnp.where(act & (c < K), m, hi)
        return (lo, hi, T, found | jnp.where(found2, 1, 0))

    lo, hi, T, found = lax.fori_loop(
        0, 32, bs_body,
        (jnp.int32(-2147483648), jnp.int32(2147483647),
         jnp.int32(0), jnp.int32(0)))
    kstar = jnp.where(found == 1, T, hi)
    c1 = count_gt(kstar)
    need = jnp.where(found == 1, 0, K - c1)
    c_sel = c1 + need

    def comp(i, off):
        k = key_v[pl.ds(i * 16, 16)]
        e = ln + i * 16
        m_gt = k > kstar
        m_eq = k == kstar
        packed = jnp.where(m_gt, 65536, 0) + jnp.where(m_eq, 1, 0)
        incl = _hillis(packed, ln) + off
        r_gt = (incl >> 16) - 1
        r_eq = (incl & 0xFFFF) - 1
        dead = BUDGET + (e & 0xFF)
        dl = jnp.where(
            m_gt, r_gt,
            jnp.where(m_eq & (r_eq < need), c1 + r_eq, dead))
        dest_v[pl.ds(i * 16, 16)] = sbase + dl * STRIDE
        return incl[15]

    lax.fori_loop(0, NVREG, comp, jnp.int32(0))
    pltpu.sync_copy(kf_v, shp.at[dest_v])

    def shift_ci(i, _):
        for u in range(4):
            o = i * 64 + u * 16
            dest_v[pl.ds(o, 16)] = dest_v[pl.ds(o, 16)] + OFS_CI
        return 0
    lax.fori_loop(0, NVREG // 4, shift_ci, 0)
    pltpu.sync_copy(if_v, shp.at[dest_v])

    for half, ofs in ((0, 0), (1, OFS_CI)):
        for cc in range(BUDGET * STRIDE // 2048):
            pltpu.sync_copy(
                shp.at[pl.ds(sbase + ofs + cc * 2048, 2048)],
                cb_f.at[pl.ds(0, 2048)])

            def diag(j, _, cc=cc, half=half):
                acc = jnp.zeros((16,), jnp.int32)
                for t in range(16):
                    v = _f2i(cb_f[pl.ds(j * 256 + t * 16, 16)])
                    acc = acc + jnp.where(ln == t,
                                          jnp.broadcast_to(v[0], (16,)), 0)
                pos = ln + cc * 128 + j * 16
                live = pos < c_sel
                if half == 0:
                    cand_k[pl.ds(DOFF + cc * 128 + j * 16, 16)] = jnp.where(
                        live, acc, _PAD_KEY)
                else:
                    cand_i[pl.ds(DOFF + cc * 128 + j * 16, 16)] = jnp.where(
                        live, acc, N + (pos & 31))
                return 0
            lax.fori_loop(0, 8, diag, 0)

    for klog in range(1, 11):
        for dlog in range(klog - 1, -1, -1):
            dd = 1 << dlog
            if dlog >= 4:
                h = dd >> 4

                def cross(t, _, h=h, klog=klog):
                    m = ((t & ~(h - 1)) << 1) | (t & (h - 1))
                    p = m + h
                    ka = cand_k[pl.ds(DOFF + m * 16, 16)]
                    ia = cand_i[pl.ds(DOFF + m * 16, 16)]
                    kb = cand_k[pl.ds(DOFF + p * 16, 16)]
                    ib = cand_i[pl.ds(DOFF + p * 16, 16)]
                    di = ((m * 16 + ln) >> klog) & 1
                    c = _cmp_gt(ka, ia, kb, ib)
                    ta = jnp.where(c, 1, 0) == (1 - di)
                    cand_k[pl.ds(DOFF + m * 16, 16)] = jnp.where(ta, ka, kb)
                    cand_i[pl.ds(DOFF + m * 16, 16)] = jnp.where(ta, ia, ib)
                    cand_k[pl.ds(DOFF + p * 16, 16)] = jnp.where(ta, kb, ka)
                    cand_i[pl.ds(DOFF + p * 16, 16)] = jnp.where(ta, ib, ia)
                    return 0
                lax.fori_loop(0, BUDGET // 32, cross, 0)
            else:
                def intra(m, _, dd=dd, klog=klog):
                    base = DOFF + m * 16
                    k = cand_k[pl.ds(base, 16)]
                    i_ = cand_i[pl.ds(base, 16)]
                    up_sel = (ln & dd) == 0
                    pk = jnp.where(up_sel,
                                   cand_k[pl.ds(base + dd, 16)],
                                   cand_k[pl.ds(base - dd, 16)])
                    pi = jnp.where(up_sel,
                                   cand_i[pl.ds(base + dd, 16)],
                                   cand_i[pl.ds(base - dd, 16)])
                    di = ((m * 16 + ln) >> klog) & 1
                    lowi = jnp.where(up_sel, 1, 0)
                    c = _cmp_gt(k, i_, pk, pi)
                    wi = jnp.where(lowi == (1 - di), 1, 0)
                    ts = jnp.where(c, 1, 0) == wi
                    cand_k[pl.ds(base, 16)] = jnp.where(ts, k, pk)
                    cand_i[pl.ds(base, 16)] = jnp.where(ts, i_, pi)
                    return 0
                lax.fori_loop(0, BUDGET // 16, intra, 0)

    def so(j, _):
        k = cand_k[pl.ds(DOFF + j * 16, 16)]
        bits = k ^ ((k >> 31) & _MAG)
        f = lax.bitcast_convert_type(bits, jnp.float32)
        sig = 1.0 / (1.0 + jnp.exp(-f))
        slot = ln + j * 16
        scrv_v[pl.ds(j * 16, 16)] = jnp.where(slot < kval, sig, 0.0)
        return 0
    lax.fori_loop(0, KPAD // 16, so, 0)
    pltpu.sync_copy(scrv_v, scr_out.at[b])

    base = b * N
    pbase = b * (P3 * N)

    def gidx(j, _):
        ii = jnp.minimum(cand_i[pl.ds(DOFF + j * 16, 16)], N - 1)
        idxg_v[pl.ds(j * 16, 16)] = ii + base
        for p in range(P3):
            idxp_v[pl.ds(p * KPAD + j * 16, 16)] = ii + (pbase + p * N)
        return 0
    lax.fori_loop(0, KPAD // 16, gidx, 0)

    for p in range(P3):
        for q in range(KPAD // 128):
            o = p * KPAD + q * 128
            pltpu.async_copy(pos_hbm.at[idxp_v.at[pl.ds(o, 128)]],
                             pout_v.at[pl.ds(o, 128)], sem).wait()

    def pmask(j, _):
        slot = ln + (j % (KPAD // 16)) * 16
        v = pout_v[pl.ds(j * 16, 16)]
        pout_v[pl.ds(j * 16, 16)] = jnp.where(slot < kval, v, 0.0)
        return 0
    lax.fori_loop(0, P3 * KPAD // 16, pmask, 0)
    pltpu.sync_copy(pout_v, pos_out.at[b])

    for ci_ in range(KPAD // CHUNK):
        start = ci_ * CHUNK
        buf = fchunk_v if ci_ % 2 == 0 else fchunk2_v
        pltpu.async_copy(feat_hbm.at[idxg_v.at[pl.ds(start, CHUNK)]],
                         buf, sem).wait()

        @pl.when(kval < K)
        def _zero_tail(buf=buf, start=start):
            def zr(r, _):
                m = jnp.where(start + r < kval, 1.0, 0.0)
                def zcol(l, _):
                    fv = buf[r, pl.ds(l * 16, 16)]
                    buf[r, pl.ds(l * 16, 16)] = fv * m
                    return 0
                lax.fori_loop(0, D // 16, zcol, 0)
                return 0
            lax.fori_loop(0, CHUNK, zr, 0)

        pltpu.sync_copy(buf, feat_out.at[b, pl.ds(start, CHUNK)])


def _run_sc(scores, feat2d, posT):
    mesh = plsc.VectorSubcoreMesh(core_axis_name="c", subcore_axis_name="s")
    f = functools.partial(
        pl.kernel,
        mesh=mesh,
        out_type=[
            jax.ShapeDtypeStruct((B, KPAD, D), jnp.float32),
            jax.ShapeDtypeStruct((B, P3 * KPAD), jnp.float32),
            jax.ShapeDtypeStruct((B, KPAD), jnp.float32),
        ],
        scratch_types=[
            pltpu.VMEM((N,), jnp.float32),
            pltpu.VMEM((N,), jnp.int32),
            pltpu.VMEM((N,), jnp.float32),
            pltpu.VMEM((N,), jnp.float32),
            pltpu.VMEM((N,), jnp.int32),
            pltpu.VMEM((2048,), jnp.float32),
            pltpu.VMEM((BUDGET + 32,), jnp.int32),
            pltpu.VMEM((BUDGET + 32,), jnp.int32),
            pltpu.VMEM((KPAD,), jnp.int32),
            pltpu.VMEM((P3 * KPAD,), jnp.int32),
            pltpu.VMEM((CHUNK, D), jnp.float32),
            pltpu.VMEM((CHUNK, D), jnp.float32),
            pltpu.VMEM((P3 * KPAD,), jnp.float32),
            pltpu.VMEM((KPAD,), jnp.float32),
            pltpu.SMEM((16,), jnp.int32),
            pltpu.VMEM_SHARED((16 * SLAB,), jnp.float32),
            pltpu.SemaphoreType.DMA,
        ],
    )(_sc_body)
    return f(scores, feat2d, posT)


def kernel(features, positions, mask, W, b):
    scores = jnp.where(mask, (features @ W + b).squeeze(-1),
                       jnp.float32(-jnp.inf))
    feat2d = features.reshape(B * N, D)
    posT = jnp.transpose(positions, (0, 2, 1)).reshape(B * P3 * N)
    feat_res, pos_res, scr_res = _run_sc(scores, feat2d, posT)
    padded_pos = jnp.transpose(pos_res.reshape(B, P3, KPAD), (0, 2, 1))[:, :K]
    padded_scores = scr_res[:, :K]
    return (padded_pos, feat_res[:, :K], padded_scores)
